# baseline (speedup 1.0000x reference)
import functools

import jax
import jax.numpy as jnp
from jax import lax
from jax.experimental import pallas as pl
from jax.experimental.pallas import tpu as pltpu
from jax.experimental.pallas import tpu_sc as plsc

_NC = 2
_NS = 16
_NW = _NC * _NS
_LANES = 16
_TILE_R = 8


@functools.partial(jax.jit, static_argnames=("n_rows", "d", "seq_len"))
def _sc_embed(table3, idx2d, pos, *, n_rows, d, seq_len):
    b_per_w = n_rows // _NW
    rows_per_w = b_per_w // 128
    n_groups = b_per_w // _LANES
    h = d // 2

    mesh = plsc.VectorSubcoreMesh(
        core_axis_name="c", subcore_axis_name="s",
        num_cores=_NC, num_subcores=_NS)

    scratch = [
        pltpu.VMEM((rows_per_w, 128), jnp.int32),
        pltpu.VMEM((seq_len, d), jnp.float32),
    ]
    scratch += [pltpu.VMEM((_LANES,), jnp.int32)] * (2 * _LANES)
    scratch += [
        pltpu.VMEM((_LANES, _TILE_R, d), jnp.float32),
        pltpu.VMEM((_LANES, _TILE_R, d), jnp.float32),
        pltpu.VMEM((_LANES, h), jnp.float32),
        pltpu.VMEM((_LANES, h), jnp.float32),
        pltpu.VMEM((_LANES, h), jnp.float32),
        pltpu.VMEM((_LANES, h), jnp.float32),
        pltpu.SemaphoreType.DMA,
        pltpu.SemaphoreType.DMA,
        pltpu.SemaphoreType.DMA,
        pltpu.SemaphoreType.DMA,
    ]

    @functools.partial(
        pl.kernel,
        out_type=(jax.ShapeDtypeStruct((n_rows, h), jnp.float32),
                  jax.ShapeDtypeStruct((n_rows, h), jnp.float32)),
        mesh=mesh,
        scratch_types=scratch,
        compiler_params=pltpu.CompilerParams(
            use_tc_tiling_on_sc=False, needs_layout_passes=False),
    )
    def k(table_hbm, idx_hbm, pos_hbm, re_hbm, im_hbm,
          idx_v, pos_v, *rest):
        tid = (rest[:_LANES], rest[_LANES:2 * _LANES])
        (bank0, bank1, sre0, sim0, sre1, sim1,
         semb0, semb1, semw0, semw1) = rest[2 * _LANES:]
        bank = (bank0, bank1)
        stg = ((sre0, sim0), (sre1, sim1))
        semb = (semb0, semb1)
        semw = (semw0, semw1)
        wid = lax.axis_index("s") * _NC + lax.axis_index("c")
        base = wid * b_per_w
        pltpu.sync_copy(idx_hbm.at[pl.ds(wid * rows_per_w, rows_per_w), :],
                        idx_v)
        pltpu.sync_copy(pos_hbm, pos_v)

        iota = lax.iota(jnp.int32, _LANES)

        def group_vec(g):
            return idx_v[g >> 3, pl.ds((g & 7) * _LANES, _LANES)]

        int_min = jnp.int32(-2**31)

        def fire(g, b):
            vec = group_vec(g)
            for lane in range(_LANES):
                t = lax.reduce_max(
                    jnp.where(iota == lane, vec, int_min), axes=(0,))
                pltpu.async_copy(table_hbm.at[pl.ds(t >> 3, 1), :, :],
                                 bank[b].at[pl.ds(lane, 1), :, :], semb[b])

        def drain_bank(b):
            pltpu.make_async_copy(table_hbm.at[pl.ds(0, _LANES), :, :],
                                  bank[b], semb[b]).wait()

        def process(g, b):
            vec = group_vec(g)
            rvec = jnp.bitwise_and(vec, 7)
            lbase = jnp.bitwise_and(g * _LANES, seq_len - 1)
            lvec = jnp.full((_LANES,), 0, jnp.int32) + lbase + iota
            for c in range(d):
                cvec = jnp.full((_LANES,), c, jnp.int32)
                tv = plsc.load_gather(bank[b], [iota, rvec, cvec])
                pv = plsc.load_gather(pos_v, [lvec, cvec])
                sv = tv + pv
                if c < h:
                    plsc.store_scatter(stg[b][0], [iota, cvec], sv)
                else:
                    plsc.store_scatter(stg[b][1], [iota, cvec - h], sv)

        def stage_out(g, b):
            dst = pl.ds(base + g * _LANES, _LANES)
            pltpu.async_copy(stg[b][0], re_hbm.at[dst, :], semw[b])
            pltpu.async_copy(stg[b][1], im_hbm.at[dst, :], semw[b])

        def drain_stage(b):
            for sref in (stg[b][0], stg[b][1]):
                pltpu.make_async_copy(
                    sref, re_hbm.at[pl.ds(0, _LANES), :], semw[b]).wait()

        fire(0, 0)

        def body(m, _):
            g0 = m * 2
            fire(g0 + 1, 1)
            drain_bank(0)

            @pl.when(m >= 1)
            def _():
                drain_stage(0)

            process(g0, 0)
            stage_out(g0, 0)

            @pl.when(m < n_groups // 2 - 1)
            def _():
                fire(g0 + 2, 0)

            drain_bank(1)

            @pl.when(m >= 1)
            def _():
                drain_stage(1)

            process(g0 + 1, 1)
            stage_out(g0 + 1, 1)
            return 0

        lax.fori_loop(0, n_groups // 2, body, 0)
        drain_stage(0)
        drain_stage(1)

    return k(table3, idx2d, pos)


def kernel(x, token_table, pos_embedding):
    B, L = x.shape
    d = token_table.shape[1]
    n_rows = B * L
    idx2d = x.reshape(n_rows // 128, 128).astype(jnp.int32)
    pos = pos_embedding[0, :L, :]
    table3 = token_table.reshape(-1, _TILE_R, d)
    re, im = _sc_embed(table3, idx2d, pos, n_rows=n_rows, d=d, seq_len=L)
    re = re.reshape(B, L, d // 2)
    im = im.reshape(B, L, d // 2)
    return jax.lax.complex(re, im)


# speedup vs baseline: 2.2422x; 2.2422x over previous
import functools

import jax
import jax.numpy as jnp
from jax import lax
from jax.experimental import pallas as pl
from jax.experimental.pallas import tpu as pltpu
from jax.experimental.pallas import tpu_sc as plsc

_NC = 2
_NS = 16
_NW = _NC * _NS
_LANES = 16
_TILE_R = 8


@functools.partial(jax.jit, static_argnames=("n_rows", "d", "seq_len"))
def _sc_embed(table, idx2d, pos, *, n_rows, d, seq_len):
    b_per_w = n_rows // _NW
    rows_per_w = b_per_w // 128
    n_groups = b_per_w // _LANES
    h = d // 2
    nch = h // _LANES

    mesh = plsc.VectorSubcoreMesh(
        core_axis_name="c", subcore_axis_name="s",
        num_cores=_NC, num_subcores=_NS)

    scratch = [
        pltpu.VMEM((rows_per_w, 128), jnp.int32),
        pltpu.VMEM((seq_len, d), jnp.float32),
    ]
    scratch += [pltpu.VMEM((_TILE_R, d), jnp.float32)] * (2 * _LANES)
    scratch += [pltpu.VMEM((_LANES, h), jnp.float32)] * 4
    scratch += [pltpu.SemaphoreType.DMA] * 4

    @functools.partial(
        pl.kernel,
        out_type=(jax.ShapeDtypeStruct((n_rows, h), jnp.float32),
                  jax.ShapeDtypeStruct((n_rows, h), jnp.float32)),
        mesh=mesh,
        scratch_types=scratch,
        compiler_params=pltpu.CompilerParams(
            use_tc_tiling_on_sc=True, needs_layout_passes=False),
    )
    def k(table_hbm, idx_hbm, pos_hbm, re_hbm, im_hbm,
          idx_v, pos_v, *bufs_sems):
        bank = (bufs_sems[:_LANES], bufs_sems[_LANES:2 * _LANES])
        st = bufs_sems[2 * _LANES:2 * _LANES + 4]
        stage = ((st[0], st[1]), (st[2], st[3]))
        sems = bufs_sems[2 * _LANES + 4:2 * _LANES + 6]
        sem_w = bufs_sems[2 * _LANES + 6:2 * _LANES + 8]
        wid = lax.axis_index("s") * _NC + lax.axis_index("c")
        base = wid * b_per_w
        pltpu.sync_copy(idx_hbm.at[pl.ds(wid * rows_per_w, rows_per_w), :],
                        idx_v)
        pltpu.sync_copy(pos_hbm, pos_v)

        lanes_iota = lax.iota(jnp.int32, _LANES)
        int_min = jnp.int32(-2**31)

        def group_vec(g):
            return idx_v[g >> 3, pl.ds((g & 7) * _LANES, _LANES)]

        def extract(g):
            vec = group_vec(g)
            return tuple(
                lax.reduce_max(
                    jnp.where(lanes_iota == lane, vec, int_min), axes=(0,))
                for lane in range(_LANES))

        def fire(ts, b):
            for lane in range(_LANES):
                pltpu.async_copy(table_hbm.at[ts[lane] >> 3],
                                 bank[b][lane], sems[b])

        def drain(b):
            for lane in range(_LANES):
                pltpu.make_async_copy(table_hbm.at[0],
                                      bank[b][lane], sems[b]).wait()

        def process(ts, g, b, p):
            for lane in range(_LANES):
                r = jnp.bitwise_and(ts[lane], 7)
                lp = jnp.bitwise_and(g * _LANES + lane, seq_len - 1)
                buf = bank[b][lane]
                for c in range(nch):
                    s = pl.ds(c * _LANES, _LANES)
                    s2 = pl.ds(h + c * _LANES, _LANES)
                    stage[p][0][lane, s] = buf[r, s] + pos_v[lp, s]
                    stage[p][1][lane, s] = buf[r, s2] + pos_v[lp, s2]

        def stage_out(g, p):
            dst = pl.ds(base + g * _LANES, _LANES)
            pltpu.async_copy(stage[p][0], re_hbm.at[dst, :], sem_w[p])
            pltpu.async_copy(stage[p][1], im_hbm.at[dst, :], sem_w[p])

        def stage_drain(p):
            for sref in (stage[p][0], stage[p][1]):
                pltpu.make_async_copy(
                    sref, re_hbm.at[pl.ds(0, _LANES), :], sem_w[p]).wait()

        ts0 = extract(0)
        fire(ts0, 0)

        def body(m, carry):
            g0 = m * 2
            ts_a = carry
            ts_b = extract(g0 + 1)
            fire(ts_b, 1)
            drain(0)

            @pl.when(m >= 1)
            def _():
                stage_drain(0)

            process(ts_a, g0, 0, 0)
            stage_out(g0, 0)

            ts_n = extract(jnp.minimum(g0 + 2, n_groups - 1))

            @pl.when(m < n_groups // 2 - 1)
            def _():
                fire(ts_n, 0)

            drain(1)

            @pl.when(m >= 1)
            def _():
                stage_drain(1)

            process(ts_b, g0 + 1, 1, 1)
            stage_out(g0 + 1, 1)
            return ts_n

        lax.fori_loop(0, n_groups // 2, body, ts0)
        stage_drain(0)
        stage_drain(1)

    return k(table, idx2d, pos)


def kernel(x, token_table, pos_embedding):
    B, L = x.shape
    d = token_table.shape[1]
    n_rows = B * L
    idx2d = x.reshape(n_rows // 128, 128).astype(jnp.int32)
    pos = pos_embedding[0, :L, :]
    table3 = token_table.reshape(-1, _TILE_R, d)
    re, im = _sc_embed(table3, idx2d, pos, n_rows=n_rows, d=d, seq_len=L)
    re = re.reshape(B, L, d // 2)
    im = im.reshape(B, L, d // 2)
    return jax.lax.complex(re, im)


# speedup vs baseline: 2.2859x; 1.0195x over previous
import functools

import jax
import jax.numpy as jnp
from jax import lax
from jax.experimental import pallas as pl
from jax.experimental.pallas import tpu as pltpu
from jax.experimental.pallas import tpu_sc as plsc

_NC = 2
_NS = 16
_NW = _NC * _NS
_LANES = 16
_TILE_R = 8


@functools.partial(jax.jit, static_argnames=("n_rows", "d", "seq_len"))
def _sc_embed(table, idx2d, pos, *, n_rows, d, seq_len):
    b_per_w = n_rows // _NW
    rows_per_w = b_per_w // 128
    n_groups = b_per_w // _LANES
    h = d // 2
    nch = h // _LANES

    mesh = plsc.VectorSubcoreMesh(
        core_axis_name="c", subcore_axis_name="s",
        num_cores=_NC, num_subcores=_NS)

    scratch = [
        pltpu.VMEM((rows_per_w, 128), jnp.int32),
        pltpu.VMEM((seq_len, d), jnp.float32),
    ]
    scratch += [pltpu.VMEM((_TILE_R, d), jnp.float32)] * (3 * _LANES)
    scratch += [pltpu.VMEM((_LANES, h), jnp.float32)] * 6
    scratch += [pltpu.SemaphoreType.DMA] * 6

    @functools.partial(
        pl.kernel,
        out_type=(jax.ShapeDtypeStruct((n_rows, h), jnp.float32),
                  jax.ShapeDtypeStruct((n_rows, h), jnp.float32)),
        mesh=mesh,
        scratch_types=scratch,
        compiler_params=pltpu.CompilerParams(
            use_tc_tiling_on_sc=True, needs_layout_passes=False),
    )
    def k(table_hbm, idx_hbm, pos_hbm, re_hbm, im_hbm,
          idx_v, pos_v, *bufs_sems):
        bank = (bufs_sems[:_LANES], bufs_sems[_LANES:2 * _LANES],
                bufs_sems[2 * _LANES:3 * _LANES])
        st = bufs_sems[3 * _LANES:3 * _LANES + 6]
        stage = ((st[0], st[1]), (st[2], st[3]), (st[4], st[5]))
        sems = bufs_sems[3 * _LANES + 6:3 * _LANES + 9]
        sem_w = bufs_sems[3 * _LANES + 9:3 * _LANES + 12]
        wid = lax.axis_index("s") * _NC + lax.axis_index("c")
        base = wid * b_per_w
        pltpu.sync_copy(idx_hbm.at[pl.ds(wid * rows_per_w, rows_per_w), :],
                        idx_v)
        pltpu.sync_copy(pos_hbm, pos_v)

        lanes_iota = lax.iota(jnp.int32, _LANES)
        int_min = jnp.int32(-2**31)

        def group_vec(g):
            return idx_v[g >> 3, pl.ds((g & 7) * _LANES, _LANES)]

        def extract(g):
            vec = group_vec(g)
            return tuple(
                lax.reduce_max(
                    jnp.where(lanes_iota == lane, vec, int_min), axes=(0,))
                for lane in range(_LANES))

        def fire(ts, b):
            for lane in range(_LANES):
                pltpu.async_copy(table_hbm.at[ts[lane] >> 3],
                                 bank[b][lane], sems[b])

        def drain(b):
            for lane in range(_LANES):
                pltpu.make_async_copy(table_hbm.at[0],
                                      bank[b][lane], sems[b]).wait()

        def process(ts, g, b, p):
            for lane in range(_LANES):
                r = jnp.bitwise_and(ts[lane], 7)
                lp = jnp.bitwise_and(g * _LANES + lane, seq_len - 1)
                buf = bank[b][lane]
                for c in range(nch):
                    s = pl.ds(c * _LANES, _LANES)
                    s2 = pl.ds(h + c * _LANES, _LANES)
                    stage[p][0][lane, s] = buf[r, s] + pos_v[lp, s]
                    stage[p][1][lane, s] = buf[r, s2] + pos_v[lp, s2]

        def stage_out(g, p):
            dst = pl.ds(base + g * _LANES, _LANES)
            pltpu.async_copy(stage[p][0], re_hbm.at[dst, :], sem_w[p])
            pltpu.async_copy(stage[p][1], im_hbm.at[dst, :], sem_w[p])

        def stage_drain(p):
            for sref in (stage[p][0], stage[p][1]):
                pltpu.make_async_copy(
                    sref, re_hbm.at[pl.ds(0, _LANES), :], sem_w[p]).wait()

        n_body = (n_groups - 1) // 3
        ts0 = extract(0)
        fire(ts0, 0)
        ts1 = extract(1)
        fire(ts1, 1)
        ts2 = extract(2)
        fire(ts2, 2)

        def one(g, b, ts_cur, m):
            drain(b)

            @pl.when(m >= 1)
            def _():
                stage_drain(b)

            process(ts_cur, g, b, b)
            stage_out(g, b)
            ts_n = extract(jnp.minimum(g + 3, n_groups - 1))

            @pl.when(g + 3 <= n_groups - 1)
            def _():
                fire(ts_n, b)

            return ts_n

        def body(m, carry):
            g0 = m * 3
            ts_a, ts_b, ts_c = carry
            ts_d = one(g0, 0, ts_a, m)
            ts_e = one(g0 + 1, 1, ts_b, m)
            ts_f = one(g0 + 2, 2, ts_c, m)
            return (ts_d, ts_e, ts_f)

        carry = lax.fori_loop(0, n_body, body, (ts0, ts1, ts2))
        drain(0)
        stage_drain(0)
        process(carry[0], n_groups - 1, 0, 0)
        stage_out(n_groups - 1, 0)
        stage_drain(1)
        stage_drain(2)
        stage_drain(0)

    return k(table, idx2d, pos)


def kernel(x, token_table, pos_embedding):
    B, L = x.shape
    d = token_table.shape[1]
    n_rows = B * L
    idx2d = x.reshape(n_rows // 128, 128).astype(jnp.int32)
    pos = pos_embedding[0, :L, :]
    table3 = token_table.reshape(-1, _TILE_R, d)
    re, im = _sc_embed(table3, idx2d, pos, n_rows=n_rows, d=d, seq_len=L)
    re = re.reshape(B, L, d // 2)
    im = im.reshape(B, L, d // 2)
    return jax.lax.complex(re, im)


# speedup vs baseline: 2.4907x; 1.0896x over previous
import functools

import jax
import jax.numpy as jnp
from jax import lax
from jax.experimental import pallas as pl
from jax.experimental.pallas import tpu as pltpu
from jax.experimental.pallas import tpu_sc as plsc

_NC = 2
_NS = 16
_NW = _NC * _NS
_LANES = 16
_TILE_R = 8


@functools.partial(jax.jit, static_argnames=("n_rows", "d", "seq_len"))
def _sc_embed(table, idx2d, pos, *, n_rows, d, seq_len):
    b_per_w = n_rows // _NW
    rows_per_w = b_per_w // 128
    n_groups = b_per_w // _LANES
    h = d // 2
    nch = h // _LANES

    mesh = plsc.VectorSubcoreMesh(
        core_axis_name="c", subcore_axis_name="s",
        num_cores=_NC, num_subcores=_NS)

    scratch = [
        pltpu.VMEM((rows_per_w, 128), jnp.int32),
        pltpu.VMEM((seq_len, d), jnp.float32),
    ]
    scratch += [pltpu.VMEM((1, d), jnp.float32)] * (3 * _LANES)
    scratch += [pltpu.VMEM((_LANES, h), jnp.float32)] * 6
    scratch += [pltpu.SemaphoreType.DMA] * 6

    @functools.partial(
        pl.kernel,
        out_type=(jax.ShapeDtypeStruct((n_rows, h), jnp.float32),
                  jax.ShapeDtypeStruct((n_rows, h), jnp.float32)),
        mesh=mesh,
        scratch_types=scratch,
        compiler_params=pltpu.CompilerParams(
            use_tc_tiling_on_sc=True, needs_layout_passes=False),
    )
    def k(table_hbm, idx_hbm, pos_hbm, re_hbm, im_hbm,
          idx_v, pos_v, *bufs_sems):
        bank = (bufs_sems[:_LANES], bufs_sems[_LANES:2 * _LANES],
                bufs_sems[2 * _LANES:3 * _LANES])
        st = bufs_sems[3 * _LANES:3 * _LANES + 6]
        stage = ((st[0], st[1]), (st[2], st[3]), (st[4], st[5]))
        sems = bufs_sems[3 * _LANES + 6:3 * _LANES + 9]
        sem_w = bufs_sems[3 * _LANES + 9:3 * _LANES + 12]
        wid = lax.axis_index("s") * _NC + lax.axis_index("c")
        base = wid * b_per_w
        pltpu.sync_copy(idx_hbm.at[pl.ds(wid * rows_per_w, rows_per_w), :],
                        idx_v)
        pltpu.sync_copy(pos_hbm, pos_v)

        lanes_iota = lax.iota(jnp.int32, _LANES)
        int_min = jnp.int32(-2**31)

        def group_vec(g):
            return idx_v[g >> 3, pl.ds((g & 7) * _LANES, _LANES)]

        def extract(g):
            vec = group_vec(g)
            return tuple(
                lax.reduce_max(
                    jnp.where(lanes_iota == lane, vec, int_min), axes=(0,))
                for lane in range(_LANES))

        def fire(ts, b):
            for lane in range(_LANES):
                t = ts[lane]
                pltpu.async_copy(
                    table_hbm.at[t >> 3,
                                 pl.ds(jnp.bitwise_and(t, _TILE_R - 1), 1), :],
                    bank[b][lane], sems[b])

        def drain(b):
            for lane in range(_LANES):
                pltpu.make_async_copy(table_hbm.at[0, pl.ds(0, 1), :],
                                      bank[b][lane], sems[b]).wait()

        def process(ts, g, b, p):
            for lane in range(_LANES):
                lp = jnp.bitwise_and(g * _LANES + lane, seq_len - 1)
                buf = bank[b][lane]
                for c in range(nch):
                    s = pl.ds(c * _LANES, _LANES)
                    s2 = pl.ds(h + c * _LANES, _LANES)
                    stage[p][0][lane, s] = buf[0, s] + pos_v[lp, s]
                    stage[p][1][lane, s] = buf[0, s2] + pos_v[lp, s2]

        def stage_out(g, p):
            dst = pl.ds(base + g * _LANES, _LANES)
            pltpu.async_copy(stage[p][0], re_hbm.at[dst, :], sem_w[p])
            pltpu.async_copy(stage[p][1], im_hbm.at[dst, :], sem_w[p])

        def stage_drain(p):
            for sref in (stage[p][0], stage[p][1]):
                pltpu.make_async_copy(
                    sref, re_hbm.at[pl.ds(0, _LANES), :], sem_w[p]).wait()

        n_body = (n_groups - 1) // 3
        ts0 = extract(0)
        fire(ts0, 0)
        ts1 = extract(1)
        fire(ts1, 1)
        ts2 = extract(2)
        fire(ts2, 2)

        def one(g, b, ts_cur, m):
            drain(b)

            @pl.when(m >= 1)
            def _():
                stage_drain(b)

            process(ts_cur, g, b, b)
            stage_out(g, b)
            ts_n = extract(jnp.minimum(g + 3, n_groups - 1))

            @pl.when(g + 3 <= n_groups - 1)
            def _():
                fire(ts_n, b)

            return ts_n

        def body(m, carry):
            g0 = m * 3
            ts_a, ts_b, ts_c = carry
            ts_d = one(g0, 0, ts_a, m)
            ts_e = one(g0 + 1, 1, ts_b, m)
            ts_f = one(g0 + 2, 2, ts_c, m)
            return (ts_d, ts_e, ts_f)

        carry = lax.fori_loop(0, n_body, body, (ts0, ts1, ts2))
        drain(0)
        stage_drain(0)
        process(carry[0], n_groups - 1, 0, 0)
        stage_out(n_groups - 1, 0)
        stage_drain(1)
        stage_drain(2)
        stage_drain(0)

    return k(table, idx2d, pos)


def kernel(x, token_table, pos_embedding):
    B, L = x.shape
    d = token_table.shape[1]
    n_rows = B * L
    idx2d = x.reshape(n_rows // 128, 128).astype(jnp.int32)
    pos = pos_embedding[0, :L, :]
    table3 = token_table.reshape(-1, _TILE_R, d)
    re, im = _sc_embed(table3, idx2d, pos, n_rows=n_rows, d=d, seq_len=L)
    re = re.reshape(B, L, d // 2)
    im = im.reshape(B, L, d // 2)
    return jax.lax.complex(re, im)


# speedup vs baseline: 2.5069x; 1.0065x over previous
import functools

import jax
import jax.numpy as jnp
from jax import lax
from jax.experimental import pallas as pl
from jax.experimental.pallas import tpu as pltpu
from jax.experimental.pallas import tpu_sc as plsc

_NC = 2
_NS = 16
_NW = _NC * _NS
_LANES = 16
_TILE_R = 8
_DEPTH = 4


@functools.partial(jax.jit, static_argnames=("n_rows", "d", "seq_len"))
def _sc_embed(table, idx2d, pos, *, n_rows, d, seq_len):
    b_per_w = n_rows // _NW
    rows_per_w = b_per_w // 128
    n_groups = b_per_w // _LANES
    h = d // 2
    nch = h // _LANES

    mesh = plsc.VectorSubcoreMesh(
        core_axis_name="c", subcore_axis_name="s",
        num_cores=_NC, num_subcores=_NS)

    scratch = [
        pltpu.VMEM((rows_per_w, 128), jnp.int32),
        pltpu.VMEM((seq_len, d), jnp.float32),
    ]
    scratch += [pltpu.VMEM((1, d), jnp.float32)] * (_DEPTH * _LANES)
    scratch += [pltpu.VMEM((_LANES, h), jnp.float32)] * (2 * _DEPTH)
    scratch += [pltpu.SemaphoreType.DMA] * (2 * _DEPTH)

    @functools.partial(
        pl.kernel,
        out_type=(jax.ShapeDtypeStruct((n_rows, h), jnp.float32),
                  jax.ShapeDtypeStruct((n_rows, h), jnp.float32)),
        mesh=mesh,
        scratch_types=scratch,
        compiler_params=pltpu.CompilerParams(
            use_tc_tiling_on_sc=True, needs_layout_passes=False),
    )
    def k(table_hbm, idx_hbm, pos_hbm, re_hbm, im_hbm,
          idx_v, pos_v, *bufs_sems):
        bank = tuple(bufs_sems[i * _LANES:(i + 1) * _LANES]
                     for i in range(_DEPTH))
        st = bufs_sems[_DEPTH * _LANES:_DEPTH * _LANES + 2 * _DEPTH]
        stage = tuple((st[2 * i], st[2 * i + 1]) for i in range(_DEPTH))
        rest = bufs_sems[_DEPTH * _LANES + 2 * _DEPTH:]
        sems = rest[:_DEPTH]
        sem_w = rest[_DEPTH:2 * _DEPTH]
        wid = lax.axis_index("s") * _NC + lax.axis_index("c")
        base = wid * b_per_w
        pltpu.sync_copy(idx_hbm.at[pl.ds(wid * rows_per_w, rows_per_w), :],
                        idx_v)
        pltpu.sync_copy(pos_hbm, pos_v)

        lanes_iota = lax.iota(jnp.int32, _LANES)
        int_min = jnp.int32(-2**31)

        def group_vec(g):
            return idx_v[g >> 3, pl.ds((g & 7) * _LANES, _LANES)]

        def extract(g):
            vec = group_vec(g)
            return tuple(
                lax.reduce_max(
                    jnp.where(lanes_iota == lane, vec, int_min), axes=(0,))
                for lane in range(_LANES))

        def fire(ts, b):
            for lane in range(_LANES):
                t = ts[lane]
                pltpu.async_copy(
                    table_hbm.at[t >> 3,
                                 pl.ds(jnp.bitwise_and(t, _TILE_R - 1), 1), :],
                    bank[b][lane], sems[b])

        def drain(b):
            for lane in range(_LANES):
                pltpu.make_async_copy(table_hbm.at[0, pl.ds(0, 1), :],
                                      bank[b][lane], sems[b]).wait()

        def process(ts, g, b, p):
            for lane in range(_LANES):
                lp = jnp.bitwise_and(g * _LANES + lane, seq_len - 1)
                buf = bank[b][lane]
                for c in range(nch):
                    s = pl.ds(c * _LANES, _LANES)
                    s2 = pl.ds(h + c * _LANES, _LANES)
                    stage[p][0][lane, s] = buf[0, s] + pos_v[lp, s]
                    stage[p][1][lane, s] = buf[0, s2] + pos_v[lp, s2]

        def stage_out(g, p):
            dst = pl.ds(base + g * _LANES, _LANES)
            pltpu.async_copy(stage[p][0], re_hbm.at[dst, :], sem_w[p])
            pltpu.async_copy(stage[p][1], im_hbm.at[dst, :], sem_w[p])

        def stage_drain(p):
            for sref in (stage[p][0], stage[p][1]):
                pltpu.make_async_copy(
                    sref, re_hbm.at[pl.ds(0, _LANES), :], sem_w[p]).wait()

        n_body = n_groups // _DEPTH
        ts_init = []
        for i in range(_DEPTH):
            tsi = extract(i)
            fire(tsi, i)
            ts_init.append(tsi)

        def one(g, b, ts_cur, m):
            drain(b)

            @pl.when(m >= 1)
            def _():
                stage_drain(b)

            process(ts_cur, g, b, b)
            stage_out(g, b)
            ts_n = extract(jnp.minimum(g + _DEPTH, n_groups - 1))

            @pl.when(g + _DEPTH <= n_groups - 1)
            def _():
                fire(ts_n, b)

            return ts_n

        def body(m, carry):
            g0 = m * _DEPTH
            return tuple(one(g0 + i, i, carry[i], m) for i in range(_DEPTH))

        lax.fori_loop(0, n_body, body, tuple(ts_init))
        for i in range(_DEPTH):
            stage_drain(i)

    return k(table, idx2d, pos)


def kernel(x, token_table, pos_embedding):
    B, L = x.shape
    d = token_table.shape[1]
    n_rows = B * L
    idx2d = x.reshape(n_rows // 128, 128).astype(jnp.int32)
    pos = pos_embedding[0, :L, :]
    table3 = token_table.reshape(-1, _TILE_R, d)
    re, im = _sc_embed(table3, idx2d, pos, n_rows=n_rows, d=d, seq_len=L)
    re = re.reshape(B, L, d // 2)
    im = im.reshape(B, L, d // 2)
    return jax.lax.complex(re, im)
